# baseline (speedup 1.0000x reference)
import functools

import jax
import jax.numpy as jnp
from jax import lax
from jax.experimental import pallas as pl
from jax.experimental.pallas import tpu as pltpu
from jax.experimental.pallas import tpu_sc as plsc

VOCAB = 100000
EMBED = 32
CTX = 20
BATCH = 1024

NUM_CORES = 2
NUM_SUBCORES = 16
NW = NUM_CORES * NUM_SUBCORES
BPW = BATCH // NW
IDX_PER_W = BPW * CTX
IDX_CHUNK = 128
N_CHUNKS = IDX_PER_W // IDX_CHUNK

VB = 2048
N_VBLK = (VOCAB + VB - 1) // VB


def _sc_pool_body(x_hbm, table_hbm, out_hbm, idx_v, rows_v, acc_v, sem):
    wid = lax.axis_index("s") * NUM_CORES + lax.axis_index("c")
    base = wid * BPW
    pltpu.sync_copy(x_hbm.at[pl.ds(base * CTX, IDX_PER_W)], idx_v)
    copies = []
    for j in range(N_CHUNKS):
        copies.append(pltpu.async_copy(
            table_hbm.at[idx_v.at[pl.ds(j * IDX_CHUNK, IDX_CHUNK)]],
            rows_v.at[pl.ds(j * IDX_CHUNK, IDX_CHUNK)],
            sem,
        ))
    for c in copies:
        c.wait()

    def body(b, carry):
        r0 = b * CTX
        acc0 = rows_v[r0, pl.ds(0, 16)]
        acc1 = rows_v[r0, pl.ds(16, 16)]
        for c in range(1, CTX):
            acc0 = acc0 + rows_v[r0 + c, pl.ds(0, 16)]
            acc1 = acc1 + rows_v[r0 + c, pl.ds(16, 16)]
        acc_v[b, pl.ds(0, 16)] = acc0
        acc_v[b, pl.ds(16, 16)] = acc1
        return carry

    lax.fori_loop(0, BPW, body, 0)
    pltpu.sync_copy(acc_v, out_hbm.at[pl.ds(base, BPW)])


def _sc_pool(x_flat, emb_table):
    mesh = plsc.VectorSubcoreMesh(core_axis_name="c", subcore_axis_name="s")
    fn = functools.partial(
        pl.kernel,
        mesh=mesh,
        out_type=jax.ShapeDtypeStruct((BATCH, EMBED), jnp.float32),
        scratch_types=[
            pltpu.VMEM((IDX_PER_W,), jnp.int32),
            pltpu.VMEM((IDX_PER_W, EMBED), jnp.float32),
            pltpu.VMEM((BPW, EMBED), jnp.float32),
            pltpu.SemaphoreType.DMA,
        ],
        compiler_params=pltpu.CompilerParams(use_tc_tiling_on_sc=False),
    )(_sc_pool_body)
    return fn(x_flat, emb_table)


def _tc_proj_body(pooled_ref, w_ref, b_ref, out_ref):
    out_ref[...] = lax.dot_general(
        pooled_ref[...], w_ref[...],
        (((1,), (1,)), ((), ())),
        preferred_element_type=jnp.float32,
    ) + b_ref[...]


def _tc_project(pooled, W, b2):
    return pl.pallas_call(
        _tc_proj_body,
        grid=(N_VBLK,),
        in_specs=[
            pl.BlockSpec((BATCH, EMBED), lambda i: (0, 0)),
            pl.BlockSpec((VB, EMBED), lambda i: (i, 0)),
            pl.BlockSpec((1, VB), lambda i: (0, i)),
        ],
        out_specs=pl.BlockSpec((BATCH, VB), lambda i: (0, i)),
        out_shape=jax.ShapeDtypeStruct((BATCH, VOCAB), jnp.float32),
    )(pooled, W, b2)


def kernel(x, emb_table, W, b):
    x_flat = x.reshape(-1).astype(jnp.int32)
    pooled = _sc_pool(x_flat, emb_table)
    return _tc_project(pooled, W, b.reshape(1, VOCAB))


# speedup vs baseline: 2.9430x; 2.9430x over previous
import functools

import jax
import jax.numpy as jnp
from jax import lax
from jax.experimental import pallas as pl
from jax.experimental.pallas import tpu as pltpu
from jax.experimental.pallas import tpu_sc as plsc

VOCAB = 100000
EMBED = 32
CTX = 20
BATCH = 1024

NUM_CORES = 2
NUM_SUBCORES = 16
NW = NUM_CORES * NUM_SUBCORES
BPW = BATCH // NW
IDX_PER_W = BPW * CTX
IDX_CHUNK = 128
N_CHUNKS = IDX_PER_W // IDX_CHUNK

VB = 2048
N_VBLK = (VOCAB + VB - 1) // VB


def _sc_pool_body(x_hbm, table_hbm, out_hbm, idx_v, rows_v, acc_v, sem):
    wid = lax.axis_index("s") * NUM_CORES + lax.axis_index("c")
    base = wid * BPW
    pltpu.sync_copy(x_hbm.at[pl.ds(base * CTX, IDX_PER_W)], idx_v)
    copies = []
    for j in range(N_CHUNKS):
        copies.append(pltpu.async_copy(
            table_hbm.at[idx_v.at[pl.ds(j * IDX_CHUNK, IDX_CHUNK)]],
            rows_v.at[pl.ds(j * IDX_CHUNK, IDX_CHUNK)],
            sem,
        ))
    for c in copies:
        c.wait()

    def body(b, carry):
        r0 = b * CTX
        acc0 = rows_v[r0, pl.ds(0, 16)]
        acc1 = rows_v[r0, pl.ds(16, 16)]
        for c in range(1, CTX):
            acc0 = acc0 + rows_v[r0 + c, pl.ds(0, 16)]
            acc1 = acc1 + rows_v[r0 + c, pl.ds(16, 16)]
        acc_v[b, pl.ds(0, 16)] = acc0
        acc_v[b, pl.ds(16, 16)] = acc1
        return carry

    lax.fori_loop(0, BPW, body, 0)
    pltpu.sync_copy(acc_v, out_hbm.at[pl.ds(base, BPW)])


def _sc_pool(x_flat, emb_table):
    mesh = plsc.VectorSubcoreMesh(core_axis_name="c", subcore_axis_name="s")
    fn = functools.partial(
        pl.kernel,
        mesh=mesh,
        out_type=jax.ShapeDtypeStruct((BATCH, EMBED), jnp.float32),
        scratch_types=[
            pltpu.VMEM((IDX_PER_W,), jnp.int32),
            pltpu.VMEM((IDX_PER_W, EMBED), jnp.float32),
            pltpu.VMEM((BPW, EMBED), jnp.float32),
            pltpu.SemaphoreType.DMA,
        ],
        compiler_params=pltpu.CompilerParams(use_tc_tiling_on_sc=False),
    )(_sc_pool_body)
    return fn(x_flat, emb_table)


def _tc_proj_body(pooled_ref, wt_ref, b_ref, out_ref):
    mm = lax.dot_general(
        wt_ref[...], pooled_ref[...],
        (((0,), (1,)), ((), ())),
        preferred_element_type=jnp.float32,
    )
    ones = jnp.ones((1, BATCH), jnp.float32)
    bias = lax.dot_general(
        b_ref[...], ones,
        (((0,), (0,)), ((), ())),
        preferred_element_type=jnp.float32,
    )
    out_ref[...] = mm + bias


def _tc_project(pooled, WT, b2):
    return pl.pallas_call(
        _tc_proj_body,
        grid=(N_VBLK,),
        in_specs=[
            pl.BlockSpec((BATCH, EMBED), lambda i: (0, 0)),
            pl.BlockSpec((EMBED, VB), lambda i: (0, i)),
            pl.BlockSpec((1, VB), lambda i: (0, i)),
        ],
        out_specs=pl.BlockSpec((VB, BATCH), lambda i: (i, 0)),
        out_shape=jax.ShapeDtypeStruct((VOCAB, BATCH), jnp.float32),
    )(pooled, WT, b2)


def kernel(x, emb_table, W, b):
    x_flat = x.reshape(-1).astype(jnp.int32)
    pooled = _sc_pool(x_flat, emb_table)
    logits_t = _tc_project(pooled, W.T, b.reshape(1, VOCAB))
    return logits_t.T
